# baseline (speedup 1.0000x reference)
import jax
import jax.numpy as jnp
from jax import lax
from jax.experimental import pallas as pl
from jax.experimental.pallas import tpu as pltpu
from jax.experimental.pallas import tpu_sc as plsc

N = 4096
MODES = 16
RES = 64
N0 = 1.5
EY_SIZE = 512
TOTAL = (N + 8) * RES
L = 16
NW = 32
ROWS_W = N // NW
HALO = 8
STAGE = ROWS_W + HALO
SEG_F = MODES * EY_SIZE
ACC_F = (STAGE - 1) * RES + EY_SIZE + RES
BODY = HALO * RES
KQ = EY_SIZE // L


def _body(u_hbm, n_hbm, ey_hbm, out_hbm,
          ustage, nstage, wstage, ey_a, ey_b, acc, sem_a, sem_b):
  cid = lax.axis_index("c")
  sid = lax.axis_index("s")
  wid = cid * 16 + sid
  r0 = wid * ROWS_W
  s_org = r0 - HALO
  s_lo = jnp.maximum(s_org, 0)
  s_hi = jnp.minimum(r0 + STAGE - HALO, N)
  nseg = s_hi - s_lo

  pltpu.sync_copy(u_hbm.at[pl.ds(s_lo * L, STAGE * L)], ustage)
  pltpu.sync_copy(n_hbm.at[pl.ds(s_lo * L, STAGE * L)], nstage)

  def wbody(li, _):
    o = li * L
    nv = nstage[pl.ds(o, L)]
    wstage[pl.ds(o, L)] = (nv * N0 / (nv + N0)) * ustage[pl.ds(o, L)]
    return 0
  lax.fori_loop(0, STAGE, wbody, 0)

  def zbody(i, _):
    acc[pl.ds(i * L, L)] = jnp.zeros((L,), jnp.float32)
    return 0
  lax.fori_loop(0, ACC_F // L, zbody, 0)

  def ey_copy(seg, buf, sem):
    return pltpu.make_async_copy(
        ey_hbm.at[pl.ds(seg * SEG_F, SEG_F)], buf, sem)

  def compute(seg, buf):
    li = seg - s_lo
    base = (seg - s_org) * RES
    wv = wstage[pl.ds(li * L, L)]
    ws = [wv.at[jnp.full((L,), m, jnp.int32)].get(mode="promise_in_bounds")
          for m in range(MODES)]
    for k in range(KQ):
      off = k * L
      v = ws[0] * buf[pl.ds(off, L)]
      for m in range(1, MODES):
        v = v + ws[m] * buf[pl.ds(m * EY_SIZE + off, L)]
      plsc.addupdate(acc.at[pl.ds(base + off, L)], v)

  ey_copy(s_lo, ey_a, sem_a).start()
  ey_copy(s_lo + 1, ey_b, sem_b).start()

  def pbody(p, _):
    sa = s_lo + 2 * p
    ey_copy(sa, ey_a, sem_a).wait()
    compute(sa, ey_a)

    @pl.when(sa + 2 < s_hi)
    def _():
      ey_copy(sa + 2, ey_a, sem_a).start()

    sb = sa + 1
    ey_copy(sb, ey_b, sem_b).wait()
    compute(sb, ey_b)

    @pl.when(sb + 2 < s_hi)
    def _():
      ey_copy(sb + 2, ey_b, sem_b).start()
    return 0
  lax.fori_loop(0, nseg // 2, pbody, 0)

  half = RES // 2

  @pl.when(wid == 0)
  def _():
    pltpu.sync_copy(acc.at[pl.ds(BODY - half, half + ROWS_W * RES)],
                    out_hbm.at[pl.ds(0, half + ROWS_W * RES)])

  @pl.when(wid == NW - 1)
  def _():
    n_tail = (ROWS_W + 7) * RES + half
    pltpu.sync_copy(acc.at[pl.ds(BODY, n_tail)],
                    out_hbm.at[pl.ds(r0 * RES + half, n_tail)])

  @pl.when((wid != 0) & (wid != NW - 1))
  def _():
    pltpu.sync_copy(acc.at[pl.ds(BODY, ROWS_W * RES)],
                    out_hbm.at[pl.ds(r0 * RES + half, ROWS_W * RES)])


_sc_call = pl.kernel(
    _body,
    out_type=jax.ShapeDtypeStruct((TOTAL,), jnp.float32),
    mesh=plsc.VectorSubcoreMesh(core_axis_name="c", subcore_axis_name="s"),
    scratch_types=[
        pltpu.VMEM((STAGE * L,), jnp.float32),
        pltpu.VMEM((STAGE * L,), jnp.float32),
        pltpu.VMEM((STAGE * L,), jnp.float32),
        pltpu.VMEM((SEG_F,), jnp.float32),
        pltpu.VMEM((SEG_F,), jnp.float32),
        pltpu.VMEM((ACC_F,), jnp.float32),
        pltpu.SemaphoreType.DMA,
        pltpu.SemaphoreType.DMA,
    ],
)


@jax.jit
def kernel(hs, U, neff, Ey):
  del hs
  return _sc_call(U.reshape(-1), neff.reshape(-1), Ey.reshape(-1))


# speedup vs baseline: 1.6231x; 1.6231x over previous
import jax
import jax.numpy as jnp
from jax import lax
from jax.experimental import pallas as pl
from jax.experimental.pallas import tpu as pltpu
from jax.experimental.pallas import tpu_sc as plsc

N = 4096
MODES = 16
RES = 64
N0 = 1.5
EY_SIZE = 512
TOTAL = (N + 8) * RES
L = 16
NW = 32
ROWS_W = N // NW
HALO = 8
STAGE = ROWS_W + HALO
SEG_F = MODES * EY_SIZE
ACC_F = (STAGE - 1) * RES + EY_SIZE + RES
BODY = HALO * RES
KQ = EY_SIZE // L


def _body(u_hbm, n_hbm, ey_hbm, out_hbm,
          ustage, nstage, wstage, ey_a, ey_b, acc, sem_a, sem_b):
  cid = lax.axis_index("c")
  sid = lax.axis_index("s")
  wid = cid * 16 + sid
  r0 = wid * ROWS_W
  s_org = r0 - HALO
  s_lo = jnp.maximum(s_org, 0)
  s_hi = jnp.minimum(r0 + STAGE - HALO, N)
  nseg = s_hi - s_lo

  pltpu.sync_copy(u_hbm.at[pl.ds(s_lo * L, STAGE * L)], ustage)
  pltpu.sync_copy(n_hbm.at[pl.ds(s_lo * L, STAGE * L)], nstage)

  def wbody(li, _):
    o = li * L
    nv = nstage[pl.ds(o, L)]
    wstage[pl.ds(o, L)] = (nv * N0 / (nv + N0)) * ustage[pl.ds(o, L)]
    return 0
  lax.fori_loop(0, STAGE, wbody, 0)

  def zbody(i, _):
    acc[pl.ds(i * L, L)] = jnp.zeros((L,), jnp.float32)
    return 0
  lax.fori_loop(0, ACC_F // L, zbody, 0)

  def ey_copy(seg, buf, sem):
    return pltpu.make_async_copy(
        ey_hbm.at[pl.ds(seg * SEG_F, SEG_F)], buf, sem)

  def compute(seg, buf):
    li = seg - s_lo
    base = (seg - s_org) * RES
    wv = wstage[pl.ds(li * L, L)]

    def mbody(m, accs):
      wm = wv.at[jnp.full((L,), m, jnp.int32)].get(mode="promise_in_bounds")
      mo = m * EY_SIZE
      return tuple(a + wm * buf[pl.ds(mo + k * L, L)]
                   for k, a in enumerate(accs))

    zero = jnp.zeros((L,), jnp.float32)
    accs = lax.fori_loop(0, MODES, mbody, (zero,) * KQ)
    for k in range(KQ):
      plsc.addupdate(acc.at[pl.ds(base + k * L, L)], accs[k])

  ey_copy(s_lo, ey_a, sem_a).start()
  ey_copy(s_lo + 1, ey_b, sem_b).start()

  def pbody(p, _):
    sa = s_lo + 2 * p
    ey_copy(sa, ey_a, sem_a).wait()
    compute(sa, ey_a)

    @pl.when(sa + 2 < s_hi)
    def _():
      ey_copy(sa + 2, ey_a, sem_a).start()

    sb = sa + 1
    ey_copy(sb, ey_b, sem_b).wait()
    compute(sb, ey_b)

    @pl.when(sb + 2 < s_hi)
    def _():
      ey_copy(sb + 2, ey_b, sem_b).start()
    return 0
  lax.fori_loop(0, nseg // 2, pbody, 0)

  half = RES // 2

  @pl.when(wid == 0)
  def _():
    pltpu.sync_copy(acc.at[pl.ds(BODY - half, half + ROWS_W * RES)],
                    out_hbm.at[pl.ds(0, half + ROWS_W * RES)])

  @pl.when(wid == NW - 1)
  def _():
    n_tail = (ROWS_W + 7) * RES + half
    pltpu.sync_copy(acc.at[pl.ds(BODY, n_tail)],
                    out_hbm.at[pl.ds(r0 * RES + half, n_tail)])

  @pl.when((wid != 0) & (wid != NW - 1))
  def _():
    pltpu.sync_copy(acc.at[pl.ds(BODY, ROWS_W * RES)],
                    out_hbm.at[pl.ds(r0 * RES + half, ROWS_W * RES)])


_sc_call = pl.kernel(
    _body,
    out_type=jax.ShapeDtypeStruct((TOTAL,), jnp.float32),
    mesh=plsc.VectorSubcoreMesh(core_axis_name="c", subcore_axis_name="s"),
    scratch_types=[
        pltpu.VMEM((STAGE * L,), jnp.float32),
        pltpu.VMEM((STAGE * L,), jnp.float32),
        pltpu.VMEM((STAGE * L,), jnp.float32),
        pltpu.VMEM((SEG_F,), jnp.float32),
        pltpu.VMEM((SEG_F,), jnp.float32),
        pltpu.VMEM((ACC_F,), jnp.float32),
        pltpu.SemaphoreType.DMA,
        pltpu.SemaphoreType.DMA,
    ],
)


@jax.jit
def kernel(hs, U, neff, Ey):
  del hs
  return _sc_call(U.reshape(-1), neff.reshape(-1), Ey.reshape(-1))


# speedup vs baseline: 1.6298x; 1.0041x over previous
import jax
import jax.numpy as jnp
from jax import lax
from jax.experimental import pallas as pl
from jax.experimental.pallas import tpu as pltpu
from jax.experimental.pallas import tpu_sc as plsc

N = 4096
MODES = 16
RES = 64
N0 = 1.5
EY_SIZE = 512
TOTAL = (N + 8) * RES
L = 16
NW = 32
ROWS_W = N // NW
HALO = 8
STAGE = ROWS_W + HALO
SEG_F = MODES * EY_SIZE
ACC_F = (STAGE - 1) * RES + EY_SIZE + RES
BODY = HALO * RES
KQ = EY_SIZE // L


def _body(u_hbm, n_hbm, ey_hbm, out_hbm,
          ustage, nstage, wstage, ey_a, ey_b, acc, sem_a, sem_b):
  cid = lax.axis_index("c")
  sid = lax.axis_index("s")
  wid = cid * 16 + sid
  r0 = wid * ROWS_W
  s_org = r0 - HALO
  s_lo = jnp.maximum(s_org, 0)
  s_hi = jnp.minimum(r0 + STAGE - HALO, N)
  nseg = s_hi - s_lo

  pltpu.sync_copy(u_hbm.at[pl.ds(s_lo * L, STAGE * L)], ustage)
  pltpu.sync_copy(n_hbm.at[pl.ds(s_lo * L, STAGE * L)], nstage)

  def wbody(li, _):
    o = li * L
    nv = nstage[pl.ds(o, L)]
    wstage[pl.ds(o, L)] = (nv * N0 / (nv + N0)) * ustage[pl.ds(o, L)]
    return 0
  lax.fori_loop(0, STAGE, wbody, 0)

  def zbody(i, _):
    acc[pl.ds(i * L, L)] = jnp.zeros((L,), jnp.float32)
    return 0
  lax.fori_loop(0, ACC_F // L, zbody, 0)

  def ey_copy(seg, buf, sem):
    return pltpu.make_async_copy(
        ey_hbm.at[pl.ds(seg * SEG_F, SEG_F)], buf, sem)

  def compute(seg, buf):
    li = seg - s_lo
    base = (seg - s_org) * RES
    wv = wstage[pl.ds(li * L, L)]

    zero = jnp.zeros((L,), jnp.float32)
    kh = KQ // 2
    for half in range(2):
      koff = half * kh * L

      def mbody(m, accs, koff=koff):
        wm = wv.at[jnp.full((L,), m, jnp.int32)].get(mode="promise_in_bounds")
        mo = m * EY_SIZE + koff
        return tuple(a + wm * buf[pl.ds(mo + k * L, L)]
                     for k, a in enumerate(accs))

      accs = lax.fori_loop(0, MODES, mbody, (zero,) * kh)
      for k in range(kh):
        plsc.addupdate(acc.at[pl.ds(base + koff + k * L, L)], accs[k])

  ey_copy(s_lo, ey_a, sem_a).start()
  ey_copy(s_lo + 1, ey_b, sem_b).start()

  def pbody(p, _):
    sa = s_lo + 2 * p
    ey_copy(sa, ey_a, sem_a).wait()
    compute(sa, ey_a)

    @pl.when(sa + 2 < s_hi)
    def _():
      ey_copy(sa + 2, ey_a, sem_a).start()

    sb = sa + 1
    ey_copy(sb, ey_b, sem_b).wait()
    compute(sb, ey_b)

    @pl.when(sb + 2 < s_hi)
    def _():
      ey_copy(sb + 2, ey_b, sem_b).start()
    return 0
  lax.fori_loop(0, nseg // 2, pbody, 0)

  half = RES // 2

  @pl.when(wid == 0)
  def _():
    pltpu.sync_copy(acc.at[pl.ds(BODY - half, half + ROWS_W * RES)],
                    out_hbm.at[pl.ds(0, half + ROWS_W * RES)])

  @pl.when(wid == NW - 1)
  def _():
    n_tail = (ROWS_W + 7) * RES + half
    pltpu.sync_copy(acc.at[pl.ds(BODY, n_tail)],
                    out_hbm.at[pl.ds(r0 * RES + half, n_tail)])

  @pl.when((wid != 0) & (wid != NW - 1))
  def _():
    pltpu.sync_copy(acc.at[pl.ds(BODY, ROWS_W * RES)],
                    out_hbm.at[pl.ds(r0 * RES + half, ROWS_W * RES)])


_sc_call = pl.kernel(
    _body,
    out_type=jax.ShapeDtypeStruct((TOTAL,), jnp.float32),
    mesh=plsc.VectorSubcoreMesh(core_axis_name="c", subcore_axis_name="s"),
    scratch_types=[
        pltpu.VMEM((STAGE * L,), jnp.float32),
        pltpu.VMEM((STAGE * L,), jnp.float32),
        pltpu.VMEM((STAGE * L,), jnp.float32),
        pltpu.VMEM((SEG_F,), jnp.float32),
        pltpu.VMEM((SEG_F,), jnp.float32),
        pltpu.VMEM((ACC_F,), jnp.float32),
        pltpu.SemaphoreType.DMA,
        pltpu.SemaphoreType.DMA,
    ],
)


@jax.jit
def kernel(hs, U, neff, Ey):
  del hs
  return _sc_call(U.reshape(-1), neff.reshape(-1), Ey.reshape(-1))


# speedup vs baseline: 1.9204x; 1.1783x over previous
import jax
import jax.numpy as jnp
from jax import lax
from jax.experimental import pallas as pl
from jax.experimental.pallas import tpu as pltpu
from jax.experimental.pallas import tpu_sc as plsc

N = 4096
MODES = 16
RES = 64
N0 = 1.5
EY_SIZE = 512
TOTAL = (N + 8) * RES
L = 16
NW = 32
ROWS_W = N // NW
HALO = 8
STAGE = ROWS_W + HALO
SEG_F = MODES * EY_SIZE
CH = 4
ACC_F = (STAGE - 1) * RES + EY_SIZE + RES
BODY = HALO * RES
KQ = EY_SIZE // L


def _body(u_hbm, n_hbm, ey_hbm, out_hbm,
          ustage, nstage, wstage, ey_a, ey_b, acc, sem_a, sem_b):
  cid = lax.axis_index("c")
  sid = lax.axis_index("s")
  wid = cid * 16 + sid
  r0 = wid * ROWS_W
  s_org = r0 - HALO
  s_lo = jnp.maximum(s_org, 0)
  s_hi = jnp.minimum(r0 + STAGE - HALO, N)
  nseg = s_hi - s_lo

  pltpu.sync_copy(u_hbm.at[pl.ds(s_lo * L, STAGE * L)], ustage)
  pltpu.sync_copy(n_hbm.at[pl.ds(s_lo * L, STAGE * L)], nstage)

  def wbody(li, _):
    o = li * L
    nv = nstage[pl.ds(o, L)]
    wstage[pl.ds(o, L)] = (nv * N0 / (nv + N0)) * ustage[pl.ds(o, L)]
    return 0
  lax.fori_loop(0, STAGE, wbody, 0)

  def zbody(i, _):
    acc[pl.ds(i * L, L)] = jnp.zeros((L,), jnp.float32)
    return 0
  lax.fori_loop(0, ACC_F // L, zbody, 0)

  def ey_copy(seg, buf, sem):
    return pltpu.make_async_copy(
        ey_hbm.at[pl.ds(seg * SEG_F, CH * SEG_F)], buf, sem)

  def compute(seg, buf, off0):
    li = seg - s_lo
    base = (seg - s_org) * RES
    wv = wstage[pl.ds(li * L, L)]

    zero = jnp.zeros((L,), jnp.float32)
    kh = KQ // 2
    for half in range(2):
      koff = half * kh * L

      def mbody(m, accs, koff=koff):
        wm = wv.at[jnp.full((L,), m, jnp.int32)].get(mode="promise_in_bounds")
        mo = off0 + m * EY_SIZE + koff
        return tuple(a + wm * buf[pl.ds(mo + k * L, L)]
                     for k, a in enumerate(accs))

      accs = lax.fori_loop(0, MODES, mbody, (zero,) * kh)
      for k in range(kh):
        plsc.addupdate(acc.at[pl.ds(base + koff + k * L, L)], accs[k])

  ey_copy(s_lo, ey_a, sem_a).start()
  ey_copy(s_lo + CH, ey_b, sem_b).start()

  def chunk(s0c, buf):
    def cbody(c, _):
      compute(s0c + c, buf, c * SEG_F)
      return 0
    lax.fori_loop(0, CH, cbody, 0)

  def pbody(p, _):
    sa = s_lo + 2 * p * CH
    ey_copy(sa, ey_a, sem_a).wait()
    chunk(sa, ey_a)

    @pl.when(sa + 2 * CH < s_hi)
    def _():
      ey_copy(sa + 2 * CH, ey_a, sem_a).start()

    sb = sa + CH
    ey_copy(sb, ey_b, sem_b).wait()
    chunk(sb, ey_b)

    @pl.when(sb + 2 * CH < s_hi)
    def _():
      ey_copy(sb + 2 * CH, ey_b, sem_b).start()
    return 0
  lax.fori_loop(0, nseg // (2 * CH), pbody, 0)

  half = RES // 2

  @pl.when(wid == 0)
  def _():
    pltpu.sync_copy(acc.at[pl.ds(BODY - half, half + ROWS_W * RES)],
                    out_hbm.at[pl.ds(0, half + ROWS_W * RES)])

  @pl.when(wid == NW - 1)
  def _():
    n_tail = (ROWS_W + 7) * RES + half
    pltpu.sync_copy(acc.at[pl.ds(BODY, n_tail)],
                    out_hbm.at[pl.ds(r0 * RES + half, n_tail)])

  @pl.when((wid != 0) & (wid != NW - 1))
  def _():
    pltpu.sync_copy(acc.at[pl.ds(BODY, ROWS_W * RES)],
                    out_hbm.at[pl.ds(r0 * RES + half, ROWS_W * RES)])


_sc_call = pl.kernel(
    _body,
    out_type=jax.ShapeDtypeStruct((TOTAL,), jnp.float32),
    mesh=plsc.VectorSubcoreMesh(core_axis_name="c", subcore_axis_name="s"),
    scratch_types=[
        pltpu.VMEM((STAGE * L,), jnp.float32),
        pltpu.VMEM((STAGE * L,), jnp.float32),
        pltpu.VMEM((STAGE * L,), jnp.float32),
        pltpu.VMEM((CH * SEG_F,), jnp.float32),
        pltpu.VMEM((CH * SEG_F,), jnp.float32),
        pltpu.VMEM((ACC_F,), jnp.float32),
        pltpu.SemaphoreType.DMA,
        pltpu.SemaphoreType.DMA,
    ],
)


@jax.jit
def kernel(hs, U, neff, Ey):
  del hs
  return _sc_call(U.reshape(-1), neff.reshape(-1), Ey.reshape(-1))


# speedup vs baseline: 3.9711x; 2.0679x over previous
import jax
import jax.numpy as jnp
from jax import lax
from jax.experimental import pallas as pl
from jax.experimental.pallas import tpu as pltpu
from jax.experimental.pallas import tpu_sc as plsc

N = 4096
MODES = 16
RES = 64
N0 = 1.5
EY_SIZE = 512
TOTAL = (N + 8) * RES
L = 16
NW = 32
ROWS_W = N // NW
HALO = 8
STAGE = ROWS_W + HALO
SEG_F = MODES * EY_SIZE
CH = 4
ACC_F = (STAGE - 1) * RES + EY_SIZE + RES
BODY = HALO * RES
KQ = EY_SIZE // L


def _body(u_hbm, n_hbm, ey_hbm, out_hbm,
          ustage, nstage, wstage, ey_a, ey_b, acc, sem_a, sem_b):
  cid = lax.axis_index("c")
  sid = lax.axis_index("s")
  wid = cid * 16 + sid
  r0 = wid * ROWS_W
  s_org = r0 - HALO
  s_lo = jnp.maximum(s_org, 0)
  s_hi = jnp.minimum(r0 + STAGE - HALO, N)
  nseg = s_hi - s_lo

  pltpu.sync_copy(u_hbm.at[pl.ds(s_lo * L, STAGE * L)], ustage)
  pltpu.sync_copy(n_hbm.at[pl.ds(s_lo * L, STAGE * L)], nstage)

  def wbody(li, _):
    o = li * L
    nv = nstage[pl.ds(o, L)]
    wstage[pl.ds(o, L)] = (nv * N0 / (nv + N0)) * ustage[pl.ds(o, L)]
    return 0
  lax.fori_loop(0, STAGE, wbody, 0)

  def zbody(i, _):
    acc[pl.ds(i * L, L)] = jnp.zeros((L,), jnp.float32)
    return 0
  lax.fori_loop(0, ACC_F // L, zbody, 0)

  def ey_copy(seg, buf, sem):
    return pltpu.make_async_copy(
        ey_hbm.at[pl.ds(seg * SEG_F, CH * SEG_F)], buf, sem)

  def compute(seg, buf, off0):
    li = seg - s_lo
    base = (seg - s_org) * RES
    wv = wstage[pl.ds(li * L, L)]

    zero = jnp.zeros((L,), jnp.float32)
    kh = KQ // 2
    for half in range(2):
      koff = half * kh * L

      def mbody(m, accs, koff=koff):
        wm = wv.at[jnp.full((L,), m, jnp.int32)].get(mode="promise_in_bounds")
        mo = off0 + m * EY_SIZE + koff
        return tuple(a + wm * buf[pl.ds(mo + k * L, L)]
                     for k, a in enumerate(accs))

      accs = lax.fori_loop(0, MODES, mbody, (zero,) * kh)
      for k in range(kh):
        plsc.addupdate(acc.at[pl.ds(base + koff + k * L, L)], accs[k])

  ey_copy(s_lo, ey_a, sem_a).start()
  ey_copy(s_lo + CH, ey_b, sem_b).start()

  def chunk(s0c, buf):
    def cbody(c, _):
      compute(s0c + c, buf, c * SEG_F)
      return 0
    lax.fori_loop(0, CH, cbody, 0)

  def pbody(p, _):
    sa = s_lo + 2 * p * CH
    ey_copy(sa, ey_a, sem_a).wait()
    chunk(sa, ey_a)

    @pl.when(sa + 2 * CH < s_hi)
    def _():
      ey_copy(sa + 2 * CH, ey_a, sem_a).start()

    sb = sa + CH
    ey_copy(sb, ey_b, sem_b).wait()
    chunk(sb, ey_b)

    @pl.when(sb + 2 * CH < s_hi)
    def _():
      ey_copy(sb + 2 * CH, ey_b, sem_b).start()
    return 0
  lax.fori_loop(0, nseg // (2 * CH), pbody, 0)

  half = RES // 2

  @pl.when(wid == 0)
  def _():
    pltpu.sync_copy(acc.at[pl.ds(BODY - half, half + ROWS_W * RES)],
                    out_hbm.at[pl.ds(0, half + ROWS_W * RES)])

  @pl.when(wid == NW - 1)
  def _():
    n_tail = (ROWS_W + 7) * RES + half
    pltpu.sync_copy(acc.at[pl.ds(BODY, n_tail)],
                    out_hbm.at[pl.ds(r0 * RES + half, n_tail)])

  @pl.when((wid != 0) & (wid != NW - 1))
  def _():
    pltpu.sync_copy(acc.at[pl.ds(BODY, ROWS_W * RES)],
                    out_hbm.at[pl.ds(r0 * RES + half, ROWS_W * RES)])


_sc_call = pl.kernel(
    _body,
    out_type=jax.ShapeDtypeStruct((TOTAL,), jnp.float32),
    mesh=plsc.VectorSubcoreMesh(core_axis_name="c", subcore_axis_name="s"),
    scratch_types=[
        pltpu.VMEM((STAGE * L,), jnp.float32),
        pltpu.VMEM((STAGE * L,), jnp.float32),
        pltpu.VMEM((STAGE * L,), jnp.float32),
        pltpu.VMEM((CH * SEG_F,), jnp.float32),
        pltpu.VMEM((CH * SEG_F,), jnp.float32),
        pltpu.VMEM((ACC_F,), jnp.float32),
        pltpu.SemaphoreType.DMA,
        pltpu.SemaphoreType.DMA,
    ],
)


TC_B = 64
TC_G = N // TC_B + 1


def _tc_body(u_ref, n_ref, ey_ref, out_ref, carry):
  g = pl.program_id(0)

  @pl.when(g == 0)
  def _():
    carry[...] = jnp.zeros((8, RES), jnp.float32)

  @pl.when(g < TC_G - 1)
  def _():
    eta = n_ref[...] * N0 / (n_ref[...] + N0)
    w = eta * u_ref[...]
    ey_sum = jnp.sum(w[:, :, None] * ey_ref[...], axis=1)
    chunks = ey_sum.reshape(TC_B, 8, RES)
    acc = jnp.zeros((TC_B + 7, RES), jnp.float32)
    for c in range(8):
      acc = acc + jnp.pad(chunks[:, c, :], ((c, 7 - c), (0, 0)))
    out = acc[:TC_B] + jnp.pad(carry[:7], ((0, TC_B - 7), (0, 0)))
    out_ref[...] = out
    carry[:7] = acc[TC_B:]

  @pl.when(g == TC_G - 1)
  def _():
    out_ref[...] = jnp.pad(carry[:7], ((0, TC_B - 7), (0, 0)))


_tc_call = pl.pallas_call(
    _tc_body,
    grid=(TC_G,),
    in_specs=[
        pl.BlockSpec((TC_B, MODES), lambda g: (jnp.minimum(g, TC_G - 2), 0)),
        pl.BlockSpec((TC_B, MODES), lambda g: (jnp.minimum(g, TC_G - 2), 0)),
        pl.BlockSpec((TC_B, MODES, EY_SIZE),
                     lambda g: (jnp.minimum(g, TC_G - 2), 0, 0)),
    ],
    out_specs=pl.BlockSpec((TC_B, RES), lambda g: (g, 0)),
    out_shape=jax.ShapeDtypeStruct((TC_G * TC_B, RES), jnp.float32),
    scratch_shapes=[pltpu.VMEM((8, RES), jnp.float32)],
)


@jax.jit
def kernel(hs, U, neff, Ey):
  del hs
  enp = _tc_call(U, neff, Ey).reshape(-1)[:TOTAL]
  return jnp.roll(enp, RES // 2)
